# baseline (speedup 1.0000x reference)
import functools

import jax
import jax.numpy as jnp
from jax import lax
from jax.experimental import pallas as pl
from jax.experimental.pallas import tpu as pltpu
from jax.experimental.pallas import tpu_sc as plsc

N = 10000
F = 256
FH = 128
E = 160000
NPAD = 10240
STRIPE = NPAD // 16
RB = 2000
GRID = N // RB

EPT = E // 16
BE = 80
NBP = 128
R = 2
NG = NBP // R
RD = 5

EPTD = E // 32
BD = 40
NBD = EPTD // BD
NGD = NBD // RD

_SC_MESH = plsc.VectorSubcoreMesh(core_axis_name="c", subcore_axis_name="s")


def _deg_body(dstd_hbm, z1_hbm, ones_hbm, out0_hbm, out1_hbm,
              acc_sh, dst_all, onesv, *ssems):
    c = lax.axis_index("c")
    s = lax.axis_index("s")
    w = c * 16 + s
    sl = pl.ds(s * STRIPE, STRIPE)
    pltpu.sync_copy(z1_hbm.at[sl], acc_sh.at[sl])
    pltpu.sync_copy(ones_hbm, onesv)
    pltpu.sync_copy(dstd_hbm.at[w], dst_all)
    plsc.subcore_barrier()

    def group(kk, carry):
        for r in range(RD):
            b = kk * RD + r

            @pl.when(kk > 0)
            def _():
                pltpu.make_async_copy(onesv, acc_sh.at[dst_all.at[0]],
                                      ssems[r]).wait()

            pltpu.async_copy(onesv, acc_sh.at[dst_all.at[b]], ssems[r],
                             add=True)
        return carry

    lax.fori_loop(0, NGD, group, 0)
    for r in range(RD):
        pltpu.make_async_copy(onesv, acc_sh.at[dst_all.at[0]], ssems[r]).wait()
    plsc.subcore_barrier()

    @pl.when(c == 0)
    def _():
        pltpu.sync_copy(acc_sh.at[sl], out0_hbm.at[sl])

    @pl.when(c == 1)
    def _():
        pltpu.sync_copy(acc_sh.at[sl], out1_hbm.at[sl])


_deg_call = pl.kernel(
    _deg_body,
    out_type=[jax.ShapeDtypeStruct((NPAD,), jnp.float32),
              jax.ShapeDtypeStruct((NPAD,), jnp.float32)],
    mesh=_SC_MESH,
    scratch_types=[
        pltpu.VMEM_SHARED((NPAD,), jnp.float32),
        pltpu.VMEM((NBD, BD), jnp.int32),
        pltpu.VMEM((BD,), jnp.float32),
    ] + [pltpu.SemaphoreType.DMA] * RD,
)


def _conv_body(hsA_hbm, hsB_hbm, src3_hbm, dst3_hbm, z2_hbm,
               outA_hbm, outB_hbm, acc_sh, srcv, dstv, rows, *sems):
    gsems = (sems[0:R], sems[R:2 * R])
    ssems = (sems[2 * R:3 * R], sems[3 * R:4 * R])
    isems = sems[4 * R:4 * R + 4]
    c = lax.axis_index("c")
    s = lax.axis_index("s")
    sl = pl.ds(s * STRIPE, STRIPE)
    pltpu.sync_copy(z2_hbm.at[sl], acc_sh.at[sl])
    plsc.subcore_barrier()

    def idx_load(g, q):
        pltpu.async_copy(src3_hbm.at[s, g], srcv.at[q], isems[q])
        pltpu.async_copy(dst3_hbm.at[s, g], dstv.at[q], isems[q])

    def idx_wait(q):
        pltpu.make_async_copy(src3_hbm.at[s, 0], srcv.at[q], isems[q]).wait()
        pltpu.make_async_copy(dst3_hbm.at[s, 0], dstv.at[q], isems[q]).wait()

    def edge_loop(hs_ref):
        def fire_gather(h, q, r):
            pltpu.async_copy(hs_ref.at[srcv.at[q, r]], rows.at[h * R + r],
                             gsems[h][r])

        for q in range(3):
            idx_load(q, q)
        for h in range(2):
            idx_wait(h)
            for r in range(R):
                fire_gather(h, h, r)

        def do_group(g, h, q):
            q2 = (q + 2) % 4
            q3 = (q + 3) % 4
            for r in range(R):
                pltpu.make_async_copy(hs_ref.at[srcv.at[q, r]],
                                      rows.at[h * R + r], gsems[h][r]).wait()
                pltpu.async_copy(rows.at[h * R + r], acc_sh.at[dstv.at[q, r]],
                                 ssems[h][r], add=True)

            @pl.when(g + 3 < NG)
            def _():
                idx_load(g + 3, q3)

            @pl.when(g + 2 < NG)
            def _():
                idx_wait(q2)
                for r in range(R):
                    pltpu.make_async_copy(rows.at[h * R + r],
                                          acc_sh.at[dstv.at[q, r]],
                                          ssems[h][r]).wait()
                    fire_gather(h, q2, r)

        def quad(jj, carry):
            g0 = 4 * jj
            do_group(g0, 0, 0)
            do_group(g0 + 1, 1, 1)
            do_group(g0 + 2, 0, 2)
            do_group(g0 + 3, 1, 3)
            return carry

        lax.fori_loop(0, NG // 4, quad, 0)
        for h in range(2):
            for r in range(R):
                pltpu.make_async_copy(rows.at[h * R + r],
                                      acc_sh.at[dstv.at[0, r]],
                                      ssems[h][r]).wait()

    @pl.when(c == 0)
    def _():
        edge_loop(hsA_hbm)

    @pl.when(c == 1)
    def _():
        edge_loop(hsB_hbm)

    plsc.subcore_barrier()

    @pl.when(c == 0)
    def _():
        pltpu.sync_copy(acc_sh.at[sl], outA_hbm.at[sl])

    @pl.when(c == 1)
    def _():
        pltpu.sync_copy(acc_sh.at[sl], outB_hbm.at[sl])


_conv_call = pl.kernel(
    _conv_body,
    out_type=[jax.ShapeDtypeStruct((NPAD, FH), jnp.float32),
              jax.ShapeDtypeStruct((NPAD, FH), jnp.float32)],
    mesh=_SC_MESH,
    scratch_types=[
        pltpu.VMEM_SHARED((NPAD, FH), jnp.float32),
        pltpu.VMEM((4, R, BE), jnp.int32),
        pltpu.VMEM((4, R, BE), jnp.int32),
        pltpu.VMEM((2 * R, BE, FH), jnp.float32),
    ] + [pltpu.SemaphoreType.DMA] * (4 * R + 4),
)


def _lrelu(v):
    return jnp.where(v >= 0, v, 0.01 * v)


def _sigmoid(v):
    return 1.0 / (1.0 + jnp.exp(-v))


def _prep_body(x_ref, w_ref, d0_ref, d1_ref,
               h1_ref, dinv_ref, dinv2_ref, hsA_ref, hsB_ref):
    h1 = jnp.dot(x_ref[...], w_ref[...], preferred_element_type=jnp.float32)
    h1_ref[...] = h1
    deg = d0_ref[...] + d1_ref[...] + 1.0
    dinv = lax.rsqrt(deg)
    dinv_ref[...] = dinv
    dinv2_ref[...] = dinv * dinv
    hs = h1 * dinv
    hsA_ref[...] = hs[:, :FH]
    hsB_ref[...] = hs[:, FH:]


def _prep_call(x, w, d0, d1):
    return pl.pallas_call(
        _prep_body,
        grid=(GRID,),
        in_specs=[pl.BlockSpec((RB, F), lambda i: (i, 0)),
                  pl.BlockSpec((F, F), lambda i: (0, 0)),
                  pl.BlockSpec((RB, 1), lambda i: (i, 0)),
                  pl.BlockSpec((RB, 1), lambda i: (i, 0))],
        out_specs=[pl.BlockSpec((RB, F), lambda i: (i, 0)),
                   pl.BlockSpec((RB, 1), lambda i: (i, 0)),
                   pl.BlockSpec((RB, 1), lambda i: (i, 0)),
                   pl.BlockSpec((RB, FH), lambda i: (i, 0)),
                   pl.BlockSpec((RB, FH), lambda i: (i, 0))],
        out_shape=[jax.ShapeDtypeStruct((N, F), jnp.float32),
                   jax.ShapeDtypeStruct((N, 1), jnp.float32),
                   jax.ShapeDtypeStruct((N, 1), jnp.float32),
                   jax.ShapeDtypeStruct((N, FH), jnp.float32),
                   jax.ShapeDtypeStruct((N, FH), jnp.float32)],
    )(x, w, d0, d1)


def _chain_body(accA_ref, accB_ref, h1_ref, dinvb_ref, dinv2b_ref,
                wl1_ref, bl1_ref, wl3e_ref, bl3e_ref, wd1_ref, bd1_ref,
                wd3_ref, bd3_ref, wg2_ref, bg1_ref,
                hs2A_ref, hs2B_ref, t2_ref):
    dv = dinvb_ref[...]
    d2 = dinv2b_ref[...]
    h1 = h1_ref[...]
    bg1 = bg1_ref[...]
    g1A = _lrelu(accA_ref[...] * dv + h1[:, :FH] * d2 + bg1[:, :FH])
    g1B = _lrelu(accB_ref[...] * dv + h1[:, FH:] * d2 + bg1[:, FH:])
    wl1 = wl1_ref[...]
    l1 = _lrelu(jnp.dot(g1A, wl1[:FH, :], preferred_element_type=jnp.float32)
                + jnp.dot(g1B, wl1[FH:, :], preferred_element_type=jnp.float32)
                + bl1_ref[...])
    z = jnp.dot(l1, wl3e_ref[...], preferred_element_type=jnp.float32) + bl3e_ref[...]
    d1 = _lrelu(jnp.dot(z, wd1_ref[...], preferred_element_type=jnp.float32) + bd1_ref[...])
    d3 = _lrelu(jnp.dot(d1, wd3_ref[...], preferred_element_type=jnp.float32) + bd3_ref[...])
    h2 = jnp.dot(d3, wg2_ref[...], preferred_element_type=jnp.float32)
    hs2A_ref[...] = h2[:, :FH] * dv
    hs2B_ref[...] = h2[:, FH:] * dv
    t2_ref[...] = jnp.concatenate([h2[:, :FH] * d2, h2[:, FH:] * d2], axis=1)


def _chain_call(accA, accB, h1, dinvb, dinv2b,
                W_l1, b_l1, W_l3e, b_l3e, W_d1, b_d1, W_d3, b_d3, W_g2, b_g1):
    L = 128
    full = lambda a, b: pl.BlockSpec((a, b), lambda i: (0, 0))
    return pl.pallas_call(
        _chain_body,
        grid=(GRID,),
        in_specs=[pl.BlockSpec((RB, FH), lambda i: (i, 0)),
                  pl.BlockSpec((RB, FH), lambda i: (i, 0)),
                  pl.BlockSpec((RB, F), lambda i: (i, 0)),
                  pl.BlockSpec((RB, 1), lambda i: (i, 0)),
                  pl.BlockSpec((RB, 1), lambda i: (i, 0)),
                  full(F, L), full(1, L),
                  full(L, L), full(1, L),
                  full(L, L), full(1, L),
                  full(L, F), full(1, F),
                  full(F, F), full(1, F)],
        out_specs=[pl.BlockSpec((RB, FH), lambda i: (i, 0)),
                   pl.BlockSpec((RB, FH), lambda i: (i, 0)),
                   pl.BlockSpec((RB, F), lambda i: (i, 0))],
        out_shape=[jax.ShapeDtypeStruct((N, FH), jnp.float32),
                   jax.ShapeDtypeStruct((N, FH), jnp.float32),
                   jax.ShapeDtypeStruct((N, F), jnp.float32)],
    )(accA, accB, h1, dinvb, dinv2b,
      W_l1, b_l1, W_l3e, b_l3e, W_d1, b_d1, W_d3, b_d3, W_g2, b_g1)


def _final_body(accA_ref, accB_ref, t2_ref, dinvb_ref, bg2_ref, out_ref):
    dv = dinvb_ref[...]
    t2 = t2_ref[...]
    bg2 = bg2_ref[...]
    oA = accA_ref[...] * dv + t2[:, :FH] + bg2[:, :FH]
    oB = accB_ref[...] * dv + t2[:, FH:] + bg2[:, FH:]
    out_ref[...] = _sigmoid(jnp.concatenate([oA, oB], axis=1))


def _final_call(accA, accB, t2, dinvb, b_g2):
    return pl.pallas_call(
        _final_body,
        grid=(GRID,),
        in_specs=[pl.BlockSpec((RB, FH), lambda i: (i, 0)),
                  pl.BlockSpec((RB, FH), lambda i: (i, 0)),
                  pl.BlockSpec((RB, F), lambda i: (i, 0)),
                  pl.BlockSpec((RB, 1), lambda i: (i, 0)),
                  pl.BlockSpec((1, F), lambda i: (0, 0))],
        out_specs=pl.BlockSpec((RB, F), lambda i: (i, 0)),
        out_shape=jax.ShapeDtypeStruct((N, F), jnp.float32),
    )(accA, accB, t2, dinvb, b_g2)


def kernel(x, edge_index, batch, W_g1, b_g1, W_l1, b_l1, W_l3e, b_l3e,
           W_d1, b_d1, W_d3, b_d3, W_g2, b_g2):
    ei = edge_index.astype(jnp.int32)
    npad_e = NBP * BE - EPT
    pad_rows = jnp.arange(npad_e, dtype=jnp.int32)[None, :]
    src_pad = jnp.broadcast_to(pad_rows, (16, npad_e))
    dst_pad = jnp.broadcast_to(N + pad_rows, (16, npad_e))
    src2 = jnp.concatenate(
        [ei[0].reshape(16, EPT), src_pad], axis=1).reshape(16, NG, R, BE)
    dst2 = jnp.concatenate(
        [ei[1].reshape(16, EPT), dst_pad], axis=1).reshape(16, NG, R, BE)
    dstd = ei[1].reshape(32, NBD, BD)
    z1 = jnp.zeros((NPAD,), jnp.float32)
    z2 = jnp.zeros((NPAD, FH), jnp.float32)
    ones = jnp.ones((BD,), jnp.float32)

    d0, d1_ = _deg_call(dstd, z1, ones)
    h1, dinvb, dinv2b, hsA, hsB = _prep_call(
        x, W_g1, d0.reshape(NPAD, 1), d1_.reshape(NPAD, 1))
    acc1A, acc1B = _conv_call(hsA, hsB, src2, dst2, z2)
    hs2A, hs2B, t2 = _chain_call(
        acc1A, acc1B, h1, dinvb, dinv2b,
        W_l1, b_l1.reshape(1, -1), W_l3e, b_l3e.reshape(1, -1),
        W_d1, b_d1.reshape(1, -1), W_d3, b_d3.reshape(1, -1),
        W_g2, b_g1.reshape(1, -1))
    acc2A, acc2B = _conv_call(hs2A, hs2B, src2, dst2, z2)
    out = _final_call(acc2A, acc2B, t2, dinvb, b_g2.reshape(1, -1))
    return out


# speedup vs baseline: 1.0434x; 1.0434x over previous
import functools

import jax
import jax.numpy as jnp
from jax import lax
from jax.experimental import pallas as pl
from jax.experimental.pallas import tpu as pltpu
from jax.experimental.pallas import tpu_sc as plsc

N = 10000
F = 256
FH = 128
E = 160000
NPAD = 10240
STRIPE = NPAD // 16
RB = 2000
GRID = N // RB

EPT = E // 16
BE = 40
NBP = 256
R = 4
NG = NBP // R
RD = 5

EPTD = E // 32
BD = 40
NBD = EPTD // BD
NGD = NBD // RD

_SC_MESH = plsc.VectorSubcoreMesh(core_axis_name="c", subcore_axis_name="s")


def _deg_body(dstd_hbm, z1_hbm, ones_hbm, out0_hbm, out1_hbm,
              acc_sh, dst_all, onesv, *ssems):
    c = lax.axis_index("c")
    s = lax.axis_index("s")
    w = c * 16 + s
    sl = pl.ds(s * STRIPE, STRIPE)
    pltpu.sync_copy(z1_hbm.at[sl], acc_sh.at[sl])
    pltpu.sync_copy(ones_hbm, onesv)
    pltpu.sync_copy(dstd_hbm.at[w], dst_all)
    plsc.subcore_barrier()

    def group(kk, carry):
        for r in range(RD):
            b = kk * RD + r

            @pl.when(kk > 0)
            def _():
                pltpu.make_async_copy(onesv, acc_sh.at[dst_all.at[0]],
                                      ssems[r]).wait()

            pltpu.async_copy(onesv, acc_sh.at[dst_all.at[b]], ssems[r],
                             add=True)
        return carry

    lax.fori_loop(0, NGD, group, 0)
    for r in range(RD):
        pltpu.make_async_copy(onesv, acc_sh.at[dst_all.at[0]], ssems[r]).wait()
    plsc.subcore_barrier()

    @pl.when(c == 0)
    def _():
        pltpu.sync_copy(acc_sh.at[sl], out0_hbm.at[sl])

    @pl.when(c == 1)
    def _():
        pltpu.sync_copy(acc_sh.at[sl], out1_hbm.at[sl])


_deg_call = pl.kernel(
    _deg_body,
    out_type=[jax.ShapeDtypeStruct((NPAD,), jnp.float32),
              jax.ShapeDtypeStruct((NPAD,), jnp.float32)],
    mesh=_SC_MESH,
    scratch_types=[
        pltpu.VMEM_SHARED((NPAD,), jnp.float32),
        pltpu.VMEM((NBD, BD), jnp.int32),
        pltpu.VMEM((BD,), jnp.float32),
    ] + [pltpu.SemaphoreType.DMA] * RD,
)


def _conv_body(hsA_hbm, hsB_hbm, src3_hbm, dst3_hbm, z2_hbm,
               outA_hbm, outB_hbm, acc_sh, srcv, dstv, rows, *sems):
    gsems = (sems[0:R], sems[R:2 * R])
    ssems = (sems[2 * R:3 * R], sems[3 * R:4 * R])
    isems = sems[4 * R:4 * R + 4]
    c = lax.axis_index("c")
    s = lax.axis_index("s")
    sl = pl.ds(s * STRIPE, STRIPE)
    pltpu.sync_copy(z2_hbm.at[sl], acc_sh.at[sl])
    plsc.subcore_barrier()

    def idx_load(g, q):
        pltpu.async_copy(src3_hbm.at[s, g], srcv.at[q], isems[q])
        pltpu.async_copy(dst3_hbm.at[s, g], dstv.at[q], isems[q])

    def idx_wait(q):
        pltpu.make_async_copy(src3_hbm.at[s, 0], srcv.at[q], isems[q]).wait()
        pltpu.make_async_copy(dst3_hbm.at[s, 0], dstv.at[q], isems[q]).wait()

    def edge_loop(hs_ref):
        def fire_gather(h, q, r):
            pltpu.async_copy(hs_ref.at[srcv.at[q, r]], rows.at[h * R + r],
                             gsems[h][r])

        for q in range(3):
            idx_load(q, q)
        for h in range(2):
            idx_wait(h)
            for r in range(R):
                fire_gather(h, h, r)

        def do_group(g, h, q):
            q2 = (q + 2) % 4
            q3 = (q + 3) % 4
            for r in range(R):
                pltpu.make_async_copy(hs_ref.at[srcv.at[q, r]],
                                      rows.at[h * R + r], gsems[h][r]).wait()
                pltpu.async_copy(rows.at[h * R + r], acc_sh.at[dstv.at[q, r]],
                                 ssems[h][r], add=True)

            @pl.when(g + 3 < NG)
            def _():
                idx_load(g + 3, q3)

            @pl.when(g + 2 < NG)
            def _():
                idx_wait(q2)
                for r in range(R):
                    pltpu.make_async_copy(rows.at[h * R + r],
                                          acc_sh.at[dstv.at[q, r]],
                                          ssems[h][r]).wait()
                    fire_gather(h, q2, r)

        def quad(jj, carry):
            g0 = 4 * jj
            do_group(g0, 0, 0)
            do_group(g0 + 1, 1, 1)
            do_group(g0 + 2, 0, 2)
            do_group(g0 + 3, 1, 3)
            return carry

        lax.fori_loop(0, NG // 4, quad, 0)
        for h in range(2):
            for r in range(R):
                pltpu.make_async_copy(rows.at[h * R + r],
                                      acc_sh.at[dstv.at[0, r]],
                                      ssems[h][r]).wait()

    @pl.when(c == 0)
    def _():
        edge_loop(hsA_hbm)

    @pl.when(c == 1)
    def _():
        edge_loop(hsB_hbm)

    plsc.subcore_barrier()

    @pl.when(c == 0)
    def _():
        pltpu.sync_copy(acc_sh.at[sl], outA_hbm.at[sl])

    @pl.when(c == 1)
    def _():
        pltpu.sync_copy(acc_sh.at[sl], outB_hbm.at[sl])


_conv_call = pl.kernel(
    _conv_body,
    out_type=[jax.ShapeDtypeStruct((NPAD, FH), jnp.float32),
              jax.ShapeDtypeStruct((NPAD, FH), jnp.float32)],
    mesh=_SC_MESH,
    scratch_types=[
        pltpu.VMEM_SHARED((NPAD, FH), jnp.float32),
        pltpu.VMEM((4, R, BE), jnp.int32),
        pltpu.VMEM((4, R, BE), jnp.int32),
        pltpu.VMEM((2 * R, BE, FH), jnp.float32),
    ] + [pltpu.SemaphoreType.DMA] * (4 * R + 4),
)


def _lrelu(v):
    return jnp.where(v >= 0, v, 0.01 * v)


def _sigmoid(v):
    return 1.0 / (1.0 + jnp.exp(-v))


def _prep_body(x_ref, w_ref, d0_ref, d1_ref,
               h1_ref, dinv_ref, dinv2_ref, hsA_ref, hsB_ref):
    h1 = jnp.dot(x_ref[...], w_ref[...], preferred_element_type=jnp.float32)
    h1_ref[...] = h1
    deg = d0_ref[...] + d1_ref[...] + 1.0
    dinv = lax.rsqrt(deg)
    dinv_ref[...] = dinv
    dinv2_ref[...] = dinv * dinv
    hs = h1 * dinv
    hsA_ref[...] = hs[:, :FH]
    hsB_ref[...] = hs[:, FH:]


def _prep_call(x, w, d0, d1):
    return pl.pallas_call(
        _prep_body,
        grid=(GRID,),
        in_specs=[pl.BlockSpec((RB, F), lambda i: (i, 0)),
                  pl.BlockSpec((F, F), lambda i: (0, 0)),
                  pl.BlockSpec((RB, 1), lambda i: (i, 0)),
                  pl.BlockSpec((RB, 1), lambda i: (i, 0))],
        out_specs=[pl.BlockSpec((RB, F), lambda i: (i, 0)),
                   pl.BlockSpec((RB, 1), lambda i: (i, 0)),
                   pl.BlockSpec((RB, 1), lambda i: (i, 0)),
                   pl.BlockSpec((RB, FH), lambda i: (i, 0)),
                   pl.BlockSpec((RB, FH), lambda i: (i, 0))],
        out_shape=[jax.ShapeDtypeStruct((N, F), jnp.float32),
                   jax.ShapeDtypeStruct((N, 1), jnp.float32),
                   jax.ShapeDtypeStruct((N, 1), jnp.float32),
                   jax.ShapeDtypeStruct((N, FH), jnp.float32),
                   jax.ShapeDtypeStruct((N, FH), jnp.float32)],
    )(x, w, d0, d1)


def _chain_body(accA_ref, accB_ref, h1_ref, dinvb_ref, dinv2b_ref,
                wl1_ref, bl1_ref, wl3e_ref, bl3e_ref, wd1_ref, bd1_ref,
                wd3_ref, bd3_ref, wg2_ref, bg1_ref,
                hs2A_ref, hs2B_ref, t2_ref):
    dv = dinvb_ref[...]
    d2 = dinv2b_ref[...]
    h1 = h1_ref[...]
    bg1 = bg1_ref[...]
    g1A = _lrelu(accA_ref[...] * dv + h1[:, :FH] * d2 + bg1[:, :FH])
    g1B = _lrelu(accB_ref[...] * dv + h1[:, FH:] * d2 + bg1[:, FH:])
    wl1 = wl1_ref[...]
    l1 = _lrelu(jnp.dot(g1A, wl1[:FH, :], preferred_element_type=jnp.float32)
                + jnp.dot(g1B, wl1[FH:, :], preferred_element_type=jnp.float32)
                + bl1_ref[...])
    z = jnp.dot(l1, wl3e_ref[...], preferred_element_type=jnp.float32) + bl3e_ref[...]
    d1 = _lrelu(jnp.dot(z, wd1_ref[...], preferred_element_type=jnp.float32) + bd1_ref[...])
    d3 = _lrelu(jnp.dot(d1, wd3_ref[...], preferred_element_type=jnp.float32) + bd3_ref[...])
    h2 = jnp.dot(d3, wg2_ref[...], preferred_element_type=jnp.float32)
    hs2A_ref[...] = h2[:, :FH] * dv
    hs2B_ref[...] = h2[:, FH:] * dv
    t2_ref[...] = jnp.concatenate([h2[:, :FH] * d2, h2[:, FH:] * d2], axis=1)


def _chain_call(accA, accB, h1, dinvb, dinv2b,
                W_l1, b_l1, W_l3e, b_l3e, W_d1, b_d1, W_d3, b_d3, W_g2, b_g1):
    L = 128
    full = lambda a, b: pl.BlockSpec((a, b), lambda i: (0, 0))
    return pl.pallas_call(
        _chain_body,
        grid=(GRID,),
        in_specs=[pl.BlockSpec((RB, FH), lambda i: (i, 0)),
                  pl.BlockSpec((RB, FH), lambda i: (i, 0)),
                  pl.BlockSpec((RB, F), lambda i: (i, 0)),
                  pl.BlockSpec((RB, 1), lambda i: (i, 0)),
                  pl.BlockSpec((RB, 1), lambda i: (i, 0)),
                  full(F, L), full(1, L),
                  full(L, L), full(1, L),
                  full(L, L), full(1, L),
                  full(L, F), full(1, F),
                  full(F, F), full(1, F)],
        out_specs=[pl.BlockSpec((RB, FH), lambda i: (i, 0)),
                   pl.BlockSpec((RB, FH), lambda i: (i, 0)),
                   pl.BlockSpec((RB, F), lambda i: (i, 0))],
        out_shape=[jax.ShapeDtypeStruct((N, FH), jnp.float32),
                   jax.ShapeDtypeStruct((N, FH), jnp.float32),
                   jax.ShapeDtypeStruct((N, F), jnp.float32)],
    )(accA, accB, h1, dinvb, dinv2b,
      W_l1, b_l1, W_l3e, b_l3e, W_d1, b_d1, W_d3, b_d3, W_g2, b_g1)


def _final_body(accA_ref, accB_ref, t2_ref, dinvb_ref, bg2_ref, out_ref):
    dv = dinvb_ref[...]
    t2 = t2_ref[...]
    bg2 = bg2_ref[...]
    oA = accA_ref[...] * dv + t2[:, :FH] + bg2[:, :FH]
    oB = accB_ref[...] * dv + t2[:, FH:] + bg2[:, FH:]
    out_ref[...] = _sigmoid(jnp.concatenate([oA, oB], axis=1))


def _final_call(accA, accB, t2, dinvb, b_g2):
    return pl.pallas_call(
        _final_body,
        grid=(GRID,),
        in_specs=[pl.BlockSpec((RB, FH), lambda i: (i, 0)),
                  pl.BlockSpec((RB, FH), lambda i: (i, 0)),
                  pl.BlockSpec((RB, F), lambda i: (i, 0)),
                  pl.BlockSpec((RB, 1), lambda i: (i, 0)),
                  pl.BlockSpec((1, F), lambda i: (0, 0))],
        out_specs=pl.BlockSpec((RB, F), lambda i: (i, 0)),
        out_shape=jax.ShapeDtypeStruct((N, F), jnp.float32),
    )(accA, accB, t2, dinvb, b_g2)


def kernel(x, edge_index, batch, W_g1, b_g1, W_l1, b_l1, W_l3e, b_l3e,
           W_d1, b_d1, W_d3, b_d3, W_g2, b_g2):
    ei = edge_index.astype(jnp.int32)
    npad_e = NBP * BE - EPT
    pad_rows = jnp.arange(npad_e, dtype=jnp.int32)[None, :]
    src_pad = jnp.broadcast_to(pad_rows, (16, npad_e))
    dst_pad = jnp.broadcast_to(N + pad_rows, (16, npad_e))
    src2 = jnp.concatenate(
        [ei[0].reshape(16, EPT), src_pad], axis=1).reshape(16, NG, R, BE)
    dst2 = jnp.concatenate(
        [ei[1].reshape(16, EPT), dst_pad], axis=1).reshape(16, NG, R, BE)
    dstd = ei[1].reshape(32, NBD, BD)
    z1 = jnp.zeros((NPAD,), jnp.float32)
    z2 = jnp.zeros((NPAD, FH), jnp.float32)
    ones = jnp.ones((BD,), jnp.float32)

    d0, d1_ = _deg_call(dstd, z1, ones)
    h1, dinvb, dinv2b, hsA, hsB = _prep_call(
        x, W_g1, d0.reshape(NPAD, 1), d1_.reshape(NPAD, 1))
    acc1A, acc1B = _conv_call(hsA, hsB, src2, dst2, z2)
    hs2A, hs2B, t2 = _chain_call(
        acc1A, acc1B, h1, dinvb, dinv2b,
        W_l1, b_l1.reshape(1, -1), W_l3e, b_l3e.reshape(1, -1),
        W_d1, b_d1.reshape(1, -1), W_d3, b_d3.reshape(1, -1),
        W_g2, b_g1.reshape(1, -1))
    acc2A, acc2B = _conv_call(hs2A, hs2B, src2, dst2, z2)
    out = _final_call(acc2A, acc2B, t2, dinvb, b_g2.reshape(1, -1))
    return out


# speedup vs baseline: 1.0769x; 1.0321x over previous
import functools

import jax
import jax.numpy as jnp
from jax import lax
from jax.experimental import pallas as pl
from jax.experimental.pallas import tpu as pltpu
from jax.experimental.pallas import tpu_sc as plsc

N = 10000
F = 256
FH = 128
E = 160000
NPAD = 10240
STRIPE = NPAD // 16
RB = 2000
GRID = N // RB

EPT = E // 16
BE = 40
NBP = 256
R = 4
NG = NBP // R
RD = 5

EPTD = E // 32
BD = 40
NBD = EPTD // BD
NGD = NBD // RD

_SC_MESH = plsc.VectorSubcoreMesh(core_axis_name="c", subcore_axis_name="s")


def _deg_body(dstd_hbm, z1_hbm, ones_hbm, out0_hbm, out1_hbm,
              acc_sh, dst_all, onesv, *ssems):
    c = lax.axis_index("c")
    s = lax.axis_index("s")
    w = c * 16 + s
    sl = pl.ds(s * STRIPE, STRIPE)
    pltpu.sync_copy(z1_hbm.at[sl], acc_sh.at[sl])
    pltpu.sync_copy(ones_hbm, onesv)
    pltpu.sync_copy(dstd_hbm.at[w], dst_all)
    plsc.subcore_barrier()

    def group(kk, carry):
        for r in range(RD):
            b = kk * RD + r

            @pl.when(kk > 0)
            def _():
                pltpu.make_async_copy(onesv, acc_sh.at[dst_all.at[0]],
                                      ssems[r]).wait()

            pltpu.async_copy(onesv, acc_sh.at[dst_all.at[b]], ssems[r],
                             add=True)
        return carry

    lax.fori_loop(0, NGD, group, 0)
    for r in range(RD):
        pltpu.make_async_copy(onesv, acc_sh.at[dst_all.at[0]], ssems[r]).wait()
    plsc.subcore_barrier()

    @pl.when(c == 0)
    def _():
        pltpu.sync_copy(acc_sh.at[sl], out0_hbm.at[sl])

    @pl.when(c == 1)
    def _():
        pltpu.sync_copy(acc_sh.at[sl], out1_hbm.at[sl])


_deg_call = pl.kernel(
    _deg_body,
    out_type=[jax.ShapeDtypeStruct((NPAD,), jnp.float32),
              jax.ShapeDtypeStruct((NPAD,), jnp.float32)],
    mesh=_SC_MESH,
    scratch_types=[
        pltpu.VMEM_SHARED((NPAD,), jnp.float32),
        pltpu.VMEM((NBD, BD), jnp.int32),
        pltpu.VMEM((BD,), jnp.float32),
    ] + [pltpu.SemaphoreType.DMA] * RD,
)


def _conv_body(hsA_hbm, hsB_hbm, src3_hbm, dst3_hbm, z2_hbm,
               outA_hbm, outB_hbm, acc_sh, srcv, dstv, rows, *sems):
    gsems = (sems[0:R], sems[R:2 * R])
    ssems = (sems[2 * R:3 * R], sems[3 * R:4 * R])
    isems = sems[4 * R:4 * R + 4]
    c = lax.axis_index("c")
    s = lax.axis_index("s")
    sl = pl.ds(s * STRIPE, STRIPE)
    pltpu.sync_copy(z2_hbm.at[sl], acc_sh.at[sl])
    plsc.subcore_barrier()

    def idx_load(g, q):
        pltpu.async_copy(src3_hbm.at[s, g], srcv.at[q], isems[q])
        pltpu.async_copy(dst3_hbm.at[s, g], dstv.at[q], isems[q])

    def idx_wait(q):
        pltpu.make_async_copy(src3_hbm.at[s, 0], srcv.at[q], isems[q]).wait()
        pltpu.make_async_copy(dst3_hbm.at[s, 0], dstv.at[q], isems[q]).wait()

    def edge_loop(hs_ref):
        def fire_gather(h, q, r):
            pltpu.async_copy(hs_ref.at[srcv.at[q, r]], rows.at[h * R + r],
                             gsems[h][r])

        for q in range(3):
            idx_load(q, q)
        for h in range(2):
            idx_wait(h)
            for r in range(R):
                fire_gather(h, h, r)

        def do_group(g, h, q):
            q2 = (q + 2) % 4
            q3 = (q + 3) % 4
            for r in range(R):
                pltpu.make_async_copy(hs_ref.at[srcv.at[q, r]],
                                      rows.at[h * R + r], gsems[h][r]).wait()
                pltpu.async_copy(rows.at[h * R + r], acc_sh.at[dstv.at[q, r]],
                                 ssems[h][r], add=True)

            @pl.when(g + 3 < NG)
            def _():
                idx_load(g + 3, q3)

            @pl.when(g + 2 < NG)
            def _():
                idx_wait(q2)
                for r in range(R):
                    pltpu.make_async_copy(rows.at[h * R + r],
                                          acc_sh.at[dstv.at[q, r]],
                                          ssems[h][r]).wait()
                    fire_gather(h, q2, r)

        def quad(jj, carry):
            g0 = 4 * jj
            do_group(g0, 0, 0)
            do_group(g0 + 1, 1, 1)
            do_group(g0 + 2, 0, 2)
            do_group(g0 + 3, 1, 3)
            return carry

        lax.fori_loop(0, NG // 4, quad, 0)
        for h in range(2):
            for r in range(R):
                pltpu.make_async_copy(rows.at[h * R + r],
                                      acc_sh.at[dstv.at[0, r]],
                                      ssems[h][r]).wait()

    @pl.when(c == 0)
    def _():
        edge_loop(hsA_hbm)

    @pl.when(c == 1)
    def _():
        edge_loop(hsB_hbm)

    plsc.subcore_barrier()

    @pl.when(c == 0)
    def _():
        pltpu.sync_copy(acc_sh.at[sl], outA_hbm.at[sl])

    @pl.when(c == 1)
    def _():
        pltpu.sync_copy(acc_sh.at[sl], outB_hbm.at[sl])


_conv_call = pl.kernel(
    _conv_body,
    out_type=[jax.ShapeDtypeStruct((NPAD, FH), jnp.float32),
              jax.ShapeDtypeStruct((NPAD, FH), jnp.float32)],
    mesh=_SC_MESH,
    scratch_types=[
        pltpu.VMEM_SHARED((NPAD, FH), jnp.float32),
        pltpu.VMEM((4, R, BE), jnp.int32),
        pltpu.VMEM((4, R, BE), jnp.int32),
        pltpu.VMEM((2 * R, BE, FH), jnp.float32),
    ] + [pltpu.SemaphoreType.DMA] * (4 * R + 4),
)


def _lrelu(v):
    return jnp.where(v >= 0, v, 0.01 * v)


def _sigmoid(v):
    return 1.0 / (1.0 + jnp.exp(-v))


def _prep_body(x_ref, w_ref, d0_ref, d1_ref,
               dinv_ref, hsA_ref, hsB_ref):
    h1 = jnp.dot(x_ref[...], w_ref[...], preferred_element_type=jnp.float32)
    deg = d0_ref[...] + d1_ref[...] + 1.0
    dinv = lax.rsqrt(deg)
    dinv_ref[...] = dinv
    hs = h1 * dinv
    hsA_ref[...] = hs[:, :FH]
    hsB_ref[...] = hs[:, FH:]


def _prep_call(x, w, d0, d1):
    return pl.pallas_call(
        _prep_body,
        grid=(GRID,),
        in_specs=[pl.BlockSpec((RB, F), lambda i: (i, 0)),
                  pl.BlockSpec((F, F), lambda i: (0, 0)),
                  pl.BlockSpec((RB, 1), lambda i: (i, 0)),
                  pl.BlockSpec((RB, 1), lambda i: (i, 0))],
        out_specs=[pl.BlockSpec((RB, 1), lambda i: (i, 0)),
                   pl.BlockSpec((RB, FH), lambda i: (i, 0)),
                   pl.BlockSpec((RB, FH), lambda i: (i, 0))],
        out_shape=[jax.ShapeDtypeStruct((N, 1), jnp.float32),
                   jax.ShapeDtypeStruct((N, FH), jnp.float32),
                   jax.ShapeDtypeStruct((N, FH), jnp.float32)],
    )(x, w, d0, d1)


def _chain_body(accA_ref, accB_ref, hsA_ref, hsB_ref, dinv_ref,
                wl1_ref, bl1_ref, wl3e_ref, bl3e_ref, wd1_ref, bd1_ref,
                wd3_ref, bd3_ref, wg2_ref, bg1_ref,
                hs2A_ref, hs2B_ref):
    dv = dinv_ref[...]
    bg1 = bg1_ref[...]
    g1A = _lrelu((accA_ref[...] + hsA_ref[...]) * dv + bg1[:, :FH])
    g1B = _lrelu((accB_ref[...] + hsB_ref[...]) * dv + bg1[:, FH:])
    wl1 = wl1_ref[...]
    l1 = _lrelu(jnp.dot(g1A, wl1[:FH, :], preferred_element_type=jnp.float32)
                + jnp.dot(g1B, wl1[FH:, :], preferred_element_type=jnp.float32)
                + bl1_ref[...])
    z = jnp.dot(l1, wl3e_ref[...], preferred_element_type=jnp.float32) + bl3e_ref[...]
    d1 = _lrelu(jnp.dot(z, wd1_ref[...], preferred_element_type=jnp.float32) + bd1_ref[...])
    d3 = _lrelu(jnp.dot(d1, wd3_ref[...], preferred_element_type=jnp.float32) + bd3_ref[...])
    h2 = jnp.dot(d3, wg2_ref[...], preferred_element_type=jnp.float32)
    hs2A_ref[...] = h2[:, :FH] * dv
    hs2B_ref[...] = h2[:, FH:] * dv


def _chain_call(accA, accB, hsA, hsB, dinv,
                W_l1, b_l1, W_l3e, b_l3e, W_d1, b_d1, W_d3, b_d3, W_g2, b_g1):
    L = 128
    full = lambda a, b: pl.BlockSpec((a, b), lambda i: (0, 0))
    return pl.pallas_call(
        _chain_body,
        grid=(GRID,),
        in_specs=[pl.BlockSpec((RB, FH), lambda i: (i, 0)),
                  pl.BlockSpec((RB, FH), lambda i: (i, 0)),
                  pl.BlockSpec((RB, FH), lambda i: (i, 0)),
                  pl.BlockSpec((RB, FH), lambda i: (i, 0)),
                  pl.BlockSpec((RB, 1), lambda i: (i, 0)),
                  full(F, L), full(1, L),
                  full(L, L), full(1, L),
                  full(L, L), full(1, L),
                  full(L, F), full(1, F),
                  full(F, F), full(1, F)],
        out_specs=[pl.BlockSpec((RB, FH), lambda i: (i, 0)),
                   pl.BlockSpec((RB, FH), lambda i: (i, 0))],
        out_shape=[jax.ShapeDtypeStruct((N, FH), jnp.float32),
                   jax.ShapeDtypeStruct((N, FH), jnp.float32)],
    )(accA, accB, hsA, hsB, dinv,
      W_l1, b_l1, W_l3e, b_l3e, W_d1, b_d1, W_d3, b_d3, W_g2, b_g1)


def _final_body(accA_ref, accB_ref, hs2A_ref, hs2B_ref, dinv_ref, bg2_ref,
                out_ref):
    dv = dinv_ref[...]
    bg2 = bg2_ref[...]
    oA = (accA_ref[...] + hs2A_ref[...]) * dv + bg2[:, :FH]
    oB = (accB_ref[...] + hs2B_ref[...]) * dv + bg2[:, FH:]
    out_ref[...] = _sigmoid(jnp.concatenate([oA, oB], axis=1))


def _final_call(accA, accB, hs2A, hs2B, dinv, b_g2):
    return pl.pallas_call(
        _final_body,
        grid=(GRID,),
        in_specs=[pl.BlockSpec((RB, FH), lambda i: (i, 0)),
                  pl.BlockSpec((RB, FH), lambda i: (i, 0)),
                  pl.BlockSpec((RB, FH), lambda i: (i, 0)),
                  pl.BlockSpec((RB, FH), lambda i: (i, 0)),
                  pl.BlockSpec((RB, 1), lambda i: (i, 0)),
                  pl.BlockSpec((1, F), lambda i: (0, 0))],
        out_specs=pl.BlockSpec((RB, F), lambda i: (i, 0)),
        out_shape=jax.ShapeDtypeStruct((N, F), jnp.float32),
    )(accA, accB, hs2A, hs2B, dinv, b_g2)


def kernel(x, edge_index, batch, W_g1, b_g1, W_l1, b_l1, W_l3e, b_l3e,
           W_d1, b_d1, W_d3, b_d3, W_g2, b_g2):
    ei = edge_index.astype(jnp.int32)
    npad_e = NBP * BE - EPT
    pad_rows = jnp.arange(npad_e, dtype=jnp.int32)[None, :]
    src_pad = jnp.broadcast_to(pad_rows, (16, npad_e))
    dst_pad = jnp.broadcast_to(N + pad_rows, (16, npad_e))
    src2 = jnp.concatenate(
        [ei[0].reshape(16, EPT), src_pad], axis=1).reshape(16, NG, R, BE)
    dst2 = jnp.concatenate(
        [ei[1].reshape(16, EPT), dst_pad], axis=1).reshape(16, NG, R, BE)
    dstd = ei[1].reshape(32, NBD, BD)
    z1 = jnp.zeros((NPAD,), jnp.float32)
    z2 = jnp.zeros((NPAD, FH), jnp.float32)
    ones = jnp.ones((BD,), jnp.float32)

    d0, d1_ = _deg_call(dstd, z1, ones)
    dinv, hsA, hsB = _prep_call(
        x, W_g1, d0.reshape(NPAD, 1), d1_.reshape(NPAD, 1))
    acc1A, acc1B = _conv_call(hsA, hsB, src2, dst2, z2)
    hs2A, hs2B = _chain_call(
        acc1A, acc1B, hsA, hsB, dinv,
        W_l1, b_l1.reshape(1, -1), W_l3e, b_l3e.reshape(1, -1),
        W_d1, b_d1.reshape(1, -1), W_d3, b_d3.reshape(1, -1),
        W_g2, b_g1.reshape(1, -1))
    acc2A, acc2B = _conv_call(hs2A, hs2B, src2, dst2, z2)
    out = _final_call(acc2A, acc2B, hs2A, hs2B, dinv, b_g2.reshape(1, -1))
    return out
